# baseline (speedup 1.0000x reference)
import functools

import jax
import jax.numpy as jnp
from jax import lax
from jax.experimental import pallas as pl
from jax.experimental.pallas import tpu as pltpu
from jax.experimental.pallas import tpu_sc as plsc

B = 64
COND = 128
CDIM = 64
NBLK = 4
D = 1000
TOTD = NBLK * D
HID = 256
IN_DIM = COND + 3 * D
CTILE = 200
NSUB = D // CTILE
NSTEP = NBLK * NSUB
_LOG2PI = 1.8378770664093453


def _sc_gather_flow(flow_W1, fidx):
    info = plsc.get_sparse_core_info()
    NC, NS = info.num_cores, info.num_subcores
    R = 256 // (NC * NS)
    mesh = plsc.VectorSubcoreMesh(core_axis_name="c", subcore_axis_name="s")

    @functools.partial(
        pl.kernel,
        mesh=mesh,
        out_type=jax.ShapeDtypeStruct((NBLK, B, HID), jnp.float32),
        scratch_types=[
            pltpu.VMEM((R,), jnp.int32),
            pltpu.VMEM((R, HID), jnp.float32),
            pltpu.SemaphoreType.DMA,
        ],
    )
    def k(fw_hbm, fidx_hbm, f_out, fi_v, fr_v, sem):
        wid = lax.axis_index("s") * NC + lax.axis_index("c")
        base = wid * R
        pltpu.sync_copy(fidx_hbm.at[pl.ds(base, R)], fi_v)
        pltpu.async_copy(fw_hbm.at[fi_v], fr_v, sem).wait()
        jj = base // B
        b0 = base % B
        pltpu.sync_copy(fr_v, f_out.at[jj, pl.ds(b0, R)])

    return k(flow_W1, fidx)


_EXT = [128, 1152, 2176, IN_DIM]


def _disc_body(mwT_ref, probs_ref, mb_ref, idxT_ref,
               cond_ref, x_ref, f_ref, w1_ref, b1_ref, w2_ref, b2_ref,
               out_ref, b0_ref, bb1_ref, b2w_ref, b3_ref, inp_ref, acc_ref,
               sems):
    ct = pl.program_id(0)
    bufs = [b0_ref, bb1_ref, b2w_ref, b3_ref]

    @pl.when(ct == 0)
    def _init():
        for jj in range(NBLK):
            pltpu.make_async_copy(
                mwT_ref.at[pl.ds(jj * D, D), pl.ds(0, _EXT[jj])],
                bufs[jj], sems.at[jj]).start()
        inp_ref[0:COND, :] = jnp.transpose(cond_ref[...])
        inp_ref[COND:, :] = jnp.zeros((3 * D, B), jnp.float32)
        acc_ref[...] = jnp.zeros((8, B), jnp.float32)

    @pl.when(ct > 0)
    def _reveal():
        k = ct - 1
        tgt = jnp.zeros((1, B), jnp.int32)
        for kk in range(3):
            tgt = jnp.where(k == kk, idxT_ref[kk:kk + 1, :], tgt)
        riota = lax.broadcasted_iota(jnp.int32, (D, B), 0)
        oh = (riota == tgt).astype(jnp.float32)
        inp_ref[pl.ds(COND + k * D, D), :] = oh

    def _step(jj):
        e = _EXT[jj]
        pltpu.make_async_copy(
            mwT_ref.at[pl.ds(jj * D, D), pl.ds(0, e)],
            bufs[jj], sems.at[jj]).wait()
        lt = jnp.dot(bufs[jj][...], inp_ref[:e, :],
                     preferred_element_type=jnp.float32)
        pT = jnp.transpose(probs_ref[:, jj * D:(jj + 1) * D])
        bT = jnp.transpose(mb_ref[:, jj * D:(jj + 1) * D])
        uT = jnp.exp(lt + bT) * pT
        tgt_j = idxT_ref[jj:jj + 1, :]
        sel = lax.broadcasted_iota(jnp.int32, (D, B), 0) == tgt_j
        nsum = jnp.sum(uT, axis=0, keepdims=True)
        ssum = jnp.sum(jnp.where(sel, uT, 0.0), axis=0, keepdims=True)
        rows = lax.broadcasted_iota(jnp.int32, (8, B), 0)
        acc_ref[...] = (acc_ref[...]
                        + jnp.where(rows == jj, nsum, 0.0)
                        + jnp.where(rows == NBLK + jj, ssum, 0.0))

    for jj in range(NBLK):
        @pl.when(ct == jj)
        def _(jj=jj):
            _step(jj)

    @pl.when(ct == NBLK - 1)
    def _finish():
        a = acc_ref[...]
        lpd = jnp.sum(jnp.log(a[NBLK:2 * NBLK, :]) - jnp.log(a[0:NBLK, :]),
                      axis=0, keepdims=True)
        cond = cond_ref[...]
        fsum = f_ref[0] + f_ref[1] + f_ref[2] + f_ref[3]
        hpre = jnp.dot(cond, w1_ref[...],
                       preferred_element_type=jnp.float32)
        h = jnp.tanh(hpre + b1_ref[...] + fsum)
        stats = jnp.dot(h, w2_ref[...], preferred_element_type=jnp.float32)
        stats = stats + b2_ref[...]
        mean = stats[:, :CDIM]
        log_std = stats[:, CDIM:]
        z = (x_ref[...] - mean) * jnp.exp(-log_std)
        lpc = jnp.sum(-0.5 * z * z - log_std - 0.5 * _LOG2PI, axis=1,
                      keepdims=True)
        out_ref[...] = lpd + lpc.reshape(1, B)


def _disc_call(mwT, discrete_probs, masked_b, idxT, condition, x, F,
               flow_W1, flow_b1, flow_W2, flow_b2):
    fixed = lambda s: pl.BlockSpec(s, lambda ct: (0,) * len(s))
    return pl.pallas_call(
        _disc_body,
        grid=(NBLK,),
        out_shape=jax.ShapeDtypeStruct((1, B), jnp.float32),
        in_specs=[
            pl.BlockSpec(memory_space=pl.ANY),
            fixed((B, TOTD)),
            fixed((1, TOTD)),
            fixed((NBLK, B)),
            fixed((B, COND)),
            fixed((B, CDIM)),
            fixed((NBLK, B, HID)),
            pl.BlockSpec((COND, HID), lambda ct: (0, 0)),
            fixed((1, HID)),
            fixed((HID, 2 * CDIM)),
            fixed((1, 2 * CDIM)),
        ],
        out_specs=pl.BlockSpec((1, B), lambda ct: (0, 0)),
        scratch_shapes=[
            pltpu.VMEM((D, _EXT[0]), jnp.float32),
            pltpu.VMEM((D, _EXT[1]), jnp.float32),
            pltpu.VMEM((D, _EXT[2]), jnp.float32),
            pltpu.VMEM((D, _EXT[3]), jnp.float32),
            pltpu.VMEM((IN_DIM, B), jnp.float32),
            pltpu.VMEM((8, B), jnp.float32),
            pltpu.SemaphoreType.DMA((4,)),
        ],
    )(mwT, discrete_probs, masked_b.reshape(1, TOTD), idxT, condition, x,
      F, flow_W1, flow_b1.reshape(1, HID), flow_W2,
      flow_b2.reshape(1, 2 * CDIM))


def kernel(indices, x, discrete_probs, condition, masked_W, masked_b,
           flow_W1, flow_b1, flow_W2, flow_b2):
    idx32 = indices.astype(jnp.int32)
    idxT = idx32.T
    offs = COND + jnp.arange(NBLK, dtype=jnp.int32)[:, None] * D
    fidx = (offs + idxT).reshape(-1)

    mwT = masked_W.T

    F = _sc_gather_flow(flow_W1, fidx)
    out = _disc_call(mwT, discrete_probs, masked_b, idxT, condition, x,
                     F, flow_W1, flow_b1, flow_W2, flow_b2)
    return out.reshape(B)


# speedup vs baseline: 1.0035x; 1.0035x over previous
import functools

import jax
import jax.numpy as jnp
from jax import lax
from jax.experimental import pallas as pl
from jax.experimental.pallas import tpu as pltpu
from jax.experimental.pallas import tpu_sc as plsc

B = 64
COND = 128
CDIM = 64
NBLK = 4
D = 1000
TOTD = NBLK * D
HID = 256
IN_DIM = COND + 3 * D
_LOG2PI = 1.8378770664093453


def _sc_gather_flow(flow_W1, fidx):
    info = plsc.get_sparse_core_info()
    NC, NS = info.num_cores, info.num_subcores
    R = 256 // (NC * NS)
    mesh = plsc.VectorSubcoreMesh(core_axis_name="c", subcore_axis_name="s")

    @functools.partial(
        pl.kernel,
        mesh=mesh,
        out_type=jax.ShapeDtypeStruct((NBLK, B, HID), jnp.float32),
        scratch_types=[
            pltpu.VMEM((R,), jnp.int32),
            pltpu.VMEM((R, HID), jnp.float32),
            pltpu.SemaphoreType.DMA,
        ],
    )
    def k(fw_hbm, fidx_hbm, f_out, fi_v, fr_v, sem):
        wid = lax.axis_index("s") * NC + lax.axis_index("c")
        base = wid * R
        pltpu.sync_copy(fidx_hbm.at[pl.ds(base, R)], fi_v)
        pltpu.async_copy(fw_hbm.at[fi_v], fr_v, sem).wait()
        jj = base // B
        b0 = base % B
        pltpu.sync_copy(fr_v, f_out.at[jj, pl.ds(b0, R)])

    return k(flow_W1, fidx)


_EXT = [128, 1152, 2176, IN_DIM]


def _disc_body(mwT_ref, probs_ref, mb_ref, idxT_ref,
               cond_ref, x_ref, f_ref, w1_ref, b1_ref, w2_ref, b2_ref,
               out_ref, b0_ref, bb1_ref, b2w_ref, b3_ref, inp_ref, acc_ref,
               sems):
    ct = pl.program_id(0)
    bufs = [b0_ref, bb1_ref, b2w_ref, b3_ref]

    @pl.when(ct == 0)
    def _init():
        for jj in range(NBLK):
            pltpu.make_async_copy(
                mwT_ref.at[pl.ds(jj * D, D), pl.ds(0, _EXT[jj])],
                bufs[jj], sems.at[jj]).start()
        inp_ref[0:COND, :] = jnp.transpose(cond_ref[...])
        inp_ref[COND:, :] = jnp.zeros((3 * D, B), jnp.float32)
        acc_ref[...] = jnp.zeros((8, B), jnp.float32)

    @pl.when(ct > 0)
    def _reveal():
        k = ct - 1
        tgt = jnp.zeros((1, B), jnp.int32)
        for kk in range(3):
            tgt = jnp.where(k == kk, idxT_ref[kk:kk + 1, :], tgt)
        riota = lax.broadcasted_iota(jnp.int32, (D, B), 0)
        oh = (riota == tgt).astype(jnp.float32)
        inp_ref[pl.ds(COND + k * D, D), :] = oh

    def _step(jj):
        e = _EXT[jj]
        pltpu.make_async_copy(
            mwT_ref.at[pl.ds(jj * D, D), pl.ds(0, e)],
            bufs[jj], sems.at[jj]).wait()
        lt = jnp.dot(bufs[jj][...], inp_ref[:e, :],
                     preferred_element_type=jnp.float32)
        pT = jnp.transpose(probs_ref[:, jj * D:(jj + 1) * D])
        bT = jnp.transpose(mb_ref[:, jj * D:(jj + 1) * D])
        uT = jnp.exp(lt + bT) * pT
        tgt_j = idxT_ref[jj:jj + 1, :]
        sel = lax.broadcasted_iota(jnp.int32, (D, B), 0) == tgt_j
        nsum = jnp.sum(uT, axis=0, keepdims=True)
        ssum = jnp.sum(jnp.where(sel, uT, 0.0), axis=0, keepdims=True)
        rows = lax.broadcasted_iota(jnp.int32, (8, B), 0)
        acc_ref[...] = (acc_ref[...]
                        + jnp.where(rows == jj, nsum, 0.0)
                        + jnp.where(rows == NBLK + jj, ssum, 0.0))

    for jj in range(NBLK):
        @pl.when(ct == jj)
        def _(jj=jj):
            _step(jj)

    @pl.when(ct == NBLK - 1)
    def _finish():
        a = acc_ref[...]
        lpd = jnp.sum(jnp.log(a[NBLK:2 * NBLK, :]) - jnp.log(a[0:NBLK, :]),
                      axis=0, keepdims=True)
        cond = cond_ref[...]
        fsum = f_ref[0] + f_ref[1] + f_ref[2] + f_ref[3]
        hpre = jnp.dot(cond, w1_ref[...],
                       preferred_element_type=jnp.float32)
        h = jnp.tanh(hpre + b1_ref[...] + fsum)
        stats = jnp.dot(h, w2_ref[...], preferred_element_type=jnp.float32)
        stats = stats + b2_ref[...]
        mean = stats[:, :CDIM]
        log_std = stats[:, CDIM:]
        z = (x_ref[...] - mean) * jnp.exp(-log_std)
        lpc = jnp.sum(-0.5 * z * z - log_std - 0.5 * _LOG2PI, axis=1,
                      keepdims=True)
        out_ref[...] = lpd + lpc.reshape(1, B)


def _disc_call(mwT, discrete_probs, masked_b, idxT, condition, x, F,
               flow_W1, flow_b1, flow_W2, flow_b2):
    fixed = lambda s: pl.BlockSpec(s, lambda ct: (0,) * len(s))
    return pl.pallas_call(
        _disc_body,
        grid=(NBLK,),
        out_shape=jax.ShapeDtypeStruct((1, B), jnp.float32),
        in_specs=[
            pl.BlockSpec(memory_space=pl.ANY),
            fixed((B, TOTD)),
            fixed((1, TOTD)),
            fixed((NBLK, B)),
            fixed((B, COND)),
            fixed((B, CDIM)),
            fixed((NBLK, B, HID)),
            pl.BlockSpec((COND, HID), lambda ct: (0, 0)),
            fixed((1, HID)),
            fixed((HID, 2 * CDIM)),
            fixed((1, 2 * CDIM)),
        ],
        out_specs=pl.BlockSpec((1, B), lambda ct: (0, 0)),
        scratch_shapes=[
            pltpu.VMEM((D, _EXT[0]), jnp.float32),
            pltpu.VMEM((D, _EXT[1]), jnp.float32),
            pltpu.VMEM((D, _EXT[2]), jnp.float32),
            pltpu.VMEM((D, _EXT[3]), jnp.float32),
            pltpu.VMEM((IN_DIM, B), jnp.float32),
            pltpu.VMEM((8, B), jnp.float32),
            pltpu.SemaphoreType.DMA((4,)),
        ],
    )(mwT, discrete_probs, masked_b.reshape(1, TOTD), idxT, condition, x,
      F, flow_W1, flow_b1.reshape(1, HID), flow_W2,
      flow_b2.reshape(1, 2 * CDIM))


def kernel(indices, x, discrete_probs, condition, masked_W, masked_b,
           flow_W1, flow_b1, flow_W2, flow_b2):
    idx32 = indices.astype(jnp.int32)
    idxT = idx32.T
    offs = COND + jnp.arange(NBLK, dtype=jnp.int32)[:, None] * D
    fidx = (offs + idxT).reshape(-1)

    mwT = masked_W.T

    F = _sc_gather_flow(flow_W1, fidx)
    out = _disc_call(mwT, discrete_probs, masked_b, idxT, condition, x,
                     F, flow_W1, flow_b1, flow_W2, flow_b2)
    return out.reshape(B)
